# baseline (speedup 1.0000x reference)
import functools
import math

import jax
import jax.numpy as jnp
from jax import lax
from jax.experimental import pallas as pl
from jax.experimental.pallas import tpu as pltpu
from jax.experimental.pallas import tpu_sc as plsc

D = 128
LN_EPS = 1e-6
EPS_FOLDED = LN_EPS / D

NC = 2
NS = 16
NW = NC * NS

GATHER = 128
CHUNK = 256
SUPER = 1024
LANES = 16


def _rsqrt_vec(v):
  i = plsc.bitcast(v, jnp.int32)
  seed = jnp.full((LANES,), 0x5F3759DF, dtype=jnp.int32)
  y = plsc.bitcast(seed - lax.shift_right_logical(i, 1), jnp.float32)
  for _ in range(3):
    y = y * (1.5 - 0.5 * v * y * y)
  return y


def _make_kernel(total_rows):
  per_w = total_rows // NW
  assert per_w % SUPER == 0

  def body(x_hbm, table_hbm, gamma_hbm, beta_hbm, out_hbm,
           idx_v, rows_v, gamma_v, beta_v, sem):
    wid = lax.axis_index("s") * NC + lax.axis_index("c")
    pltpu.sync_copy(gamma_hbm, gamma_v)
    pltpu.sync_copy(beta_hbm, beta_v)
    base_w = wid * per_w
    lane_iota = lax.iota(jnp.int32, LANES)

    UNROLL = 16

    def block_body(b, bcarry):
      row_ids = lane_iota + b * LANES

      def sum_body(jg, carry):
        s, ss = carry
        j0 = jg * UNROLL
        for jj in range(UNROLL):
          cols = jnp.full((LANES,), j0 + jj, dtype=jnp.int32)
          e = plsc.load_gather(rows_v, [row_ids, cols])
          s = s + e
          ss = ss + e * e
        return s, ss

      zeros = jnp.zeros((LANES,), dtype=jnp.float32)
      s, ss = lax.fori_loop(0, D // UNROLL, sum_body, (zeros, zeros))
      mean = s * (1.0 / D)
      var = jnp.maximum(ss * (1.0 / D) - mean * mean, 0.0)
      rstd = _rsqrt_vec(var + EPS_FOLDED)
      shift = -mean * rstd

      def norm_body(jg, carry):
        j0 = jg * UNROLL
        gvec = gamma_v[pl.ds(j0, UNROLL)]
        bvec = beta_v[pl.ds(j0, UNROLL)]
        for jj in range(UNROLL):
          cols = jnp.full((LANES,), j0 + jj, dtype=jnp.int32)
          e = plsc.load_gather(rows_v, [row_ids, cols])
          o = (e * rstd + shift) * gvec[jj] + bvec[jj]
          plsc.store_scatter(rows_v, [row_ids, cols], o)
        return carry

      lax.fori_loop(0, D // UNROLL, norm_body, 0)
      return bcarry

    def super_body(si, carry):
      sbase = base_w + si * SUPER
      xrow = pl.multiple_of(sbase // GATHER, SUPER // GATHER)
      pltpu.sync_copy(x_hbm.at[pl.ds(xrow, SUPER // GATHER)], idx_v)
      for sub in range(SUPER // CHUNK):
        base = sbase + sub * CHUNK
        cps = [
            pltpu.async_copy(
                table_hbm.at[idx_v.at[sub * (CHUNK // GATHER) + g]],
                rows_v.at[pl.ds(g * GATHER, GATHER)], sem)
            for g in range(CHUNK // GATHER)
        ]
        for cp in cps:
          cp.wait()
        lax.fori_loop(0, CHUNK // LANES, block_body, 0)
        pltpu.sync_copy(rows_v, out_hbm.at[pl.ds(base, CHUNK)])
      return carry

    lax.fori_loop(0, per_w // SUPER, super_body, 0)

  return body


@jax.jit
def kernel(x, table, gamma, beta):
  bsz, seq = x.shape
  total = bsz * seq
  x2 = x.reshape(total // GATHER, GATHER)
  run = pl.kernel(
      _make_kernel(total),
      out_type=jax.ShapeDtypeStruct((total, D), jnp.float32),
      mesh=plsc.VectorSubcoreMesh(core_axis_name="c", subcore_axis_name="s"),
      compiler_params=pltpu.CompilerParams(needs_layout_passes=False),
      scratch_types=[
          pltpu.VMEM((SUPER // GATHER, GATHER), jnp.int32),
          pltpu.VMEM((CHUNK, D), jnp.float32),
          pltpu.VMEM((D,), jnp.float32),
          pltpu.VMEM((D,), jnp.float32),
          pltpu.SemaphoreType.DMA,
      ],
  )
  out = run(x2, table, gamma, beta)
  return out.reshape(bsz, seq, D)


# speedup vs baseline: 1.4935x; 1.4935x over previous
import functools
import math

import jax
import jax.numpy as jnp
from jax import lax
from jax.experimental import pallas as pl
from jax.experimental.pallas import tpu as pltpu
from jax.experimental.pallas import tpu_sc as plsc

D = 128
LN_EPS = 1e-6
EPS_FOLDED = LN_EPS / D

NC = 2
NS = 16
NW = NC * NS

GATHER = 128
CHUNK = 256
SUPER = 1024
LANES = 16


def _rsqrt_vec(v):
  i = plsc.bitcast(v, jnp.int32)
  seed = jnp.full((LANES,), 0x5F3759DF, dtype=jnp.int32)
  y = plsc.bitcast(seed - lax.shift_right_logical(i, 1), jnp.float32)
  for _ in range(3):
    y = y * (1.5 - 0.5 * v * y * y)
  return y


def _tree_sum(vs):
  while len(vs) > 1:
    vs = [a + b for a, b in zip(vs[::2], vs[1::2])]
  return vs[0]


def _make_kernel(total_rows):
  per_w = total_rows // NW
  assert per_w % SUPER == 0
  n_chunks = per_w // CHUNK
  chunks_per_super = SUPER // CHUNK
  g_per_chunk = CHUNK // GATHER

  def body(x_hbm, table_hbm, gamma_hbm, beta_hbm, out_hbm,
           idx_v, rows_a, rows_b, gamma_v, beta_v, gsa, gsb, ssa, ssb):
    wid = lax.axis_index("s") * NC + lax.axis_index("c")
    pltpu.sync_copy(gamma_hbm, gamma_v)
    pltpu.sync_copy(beta_hbm, beta_v)
    base_w = wid * per_w
    lane_iota = lax.iota(jnp.int32, LANES)

    def fetch_idx(si):
      xrow = pl.multiple_of((base_w + si * SUPER) // GATHER, SUPER // GATHER)
      pltpu.sync_copy(x_hbm.at[pl.ds(xrow, SUPER // GATHER)], idx_v)

    def start_gather(c, rows, gsem):
      irow = lax.rem(c, chunks_per_super) * g_per_chunk
      for g in range(g_per_chunk):
        pltpu.async_copy(
            table_hbm.at[idx_v.at[irow + g]],
            rows.at[pl.ds(g * GATHER, GATHER)], gsem)

    def drain_gather(rows, gsem):
      for g in range(g_per_chunk):
        pltpu.make_async_copy(
            table_hbm.at[idx_v.at[g]],
            rows.at[pl.ds(g * GATHER, GATHER)], gsem).wait()

    def start_scatter(c, rows, ssem):
      pltpu.async_copy(rows, out_hbm.at[pl.ds(base_w + c * CHUNK, CHUNK)],
                       ssem)

    def drain_scatter(rows, ssem):
      pltpu.make_async_copy(rows, out_hbm.at[pl.ds(base_w, CHUNK)],
                            ssem).wait()

    def compute(rows):
      def block_body(blk, bcarry):
        row_ids = lane_iota + blk * LANES
        zeros = jnp.zeros((LANES,), dtype=jnp.float32)

        def sum_body(j0, carry):
          s, ss = carry
          es = []
          for jj in range(LANES):
            cols = jnp.full((LANES,), j0 + jj, dtype=jnp.int32)
            es.append(plsc.load_gather(rows, [row_ids, cols]))
          s = s + _tree_sum(es)
          ss = ss + _tree_sum([e * e for e in es])
          return s, ss

        s, ss = plsc.parallel_loop(
            0, D, LANES, unroll=2, carry=(zeros, zeros))(sum_body)
        mean = s * (1.0 / D)
        var = jnp.maximum(ss * (1.0 / D) - mean * mean, 0.0)
        rstd = _rsqrt_vec(var + EPS_FOLDED)
        shift = -mean * rstd

        def norm_body(j0):
          gvec = gamma_v[pl.ds(j0, LANES)]
          bvec = beta_v[pl.ds(j0, LANES)]
          for jj in range(LANES):
            cols = jnp.full((LANES,), j0 + jj, dtype=jnp.int32)
            e = plsc.load_gather(rows, [row_ids, cols])
            o = (e * rstd + shift) * gvec[jj] + bvec[jj]
            plsc.store_scatter(rows, [row_ids, cols], o)

        plsc.parallel_loop(0, D, LANES, unroll=2)(norm_body)
        return bcarry

      lax.fori_loop(0, CHUNK // LANES, block_body, 0)

    fetch_idx(0)
    start_gather(0, rows_a, gsa)

    def pair_body(p, carry):
      c0 = 2 * p
      c1 = c0 + 1
      c2 = c0 + 2

      @pl.when(p > 0)
      def _():
        drain_scatter(rows_b, ssb)
      start_gather(c1, rows_b, gsb)
      drain_gather(rows_a, gsa)
      compute(rows_a)
      start_scatter(c0, rows_a, ssa)

      drain_gather(rows_b, gsb)

      @pl.when(c2 < n_chunks)
      def _():
        @pl.when(lax.rem(c2, chunks_per_super) == 0)
        def _():
          fetch_idx(c2 // chunks_per_super)
        drain_scatter(rows_a, ssa)
        start_gather(c2, rows_a, gsa)
      compute(rows_b)
      start_scatter(c1, rows_b, ssb)
      return carry

    lax.fori_loop(0, n_chunks // 2, pair_body, 0)
    drain_scatter(rows_a, ssa)
    drain_scatter(rows_b, ssb)

  return body


@jax.jit
def kernel(x, table, gamma, beta):
  bsz, seq = x.shape
  total = bsz * seq
  x2 = x.reshape(total // GATHER, GATHER)
  run = pl.kernel(
      _make_kernel(total),
      out_type=jax.ShapeDtypeStruct((total, D), jnp.float32),
      mesh=plsc.VectorSubcoreMesh(core_axis_name="c", subcore_axis_name="s"),
      compiler_params=pltpu.CompilerParams(needs_layout_passes=False),
      scratch_types=[
          pltpu.VMEM((SUPER // GATHER, GATHER), jnp.int32),
          pltpu.VMEM((CHUNK, D), jnp.float32),
          pltpu.VMEM((CHUNK, D), jnp.float32),
          pltpu.VMEM((D,), jnp.float32),
          pltpu.VMEM((D,), jnp.float32),
          pltpu.SemaphoreType.DMA,
          pltpu.SemaphoreType.DMA,
          pltpu.SemaphoreType.DMA,
          pltpu.SemaphoreType.DMA,
      ],
  )
  out = run(x2, table, gamma, beta)
  return out.reshape(bsz, seq, D)


# speedup vs baseline: 12.1108x; 8.1090x over previous
import functools
import math

import jax
import jax.numpy as jnp
from jax import lax
from jax.experimental import pallas as pl
from jax.experimental.pallas import tpu as pltpu
from jax.experimental.pallas import tpu_sc as plsc

D = 128
LN_EPS = 1e-6
EPS_FOLDED = LN_EPS / D

NC = 2
NS = 16
NW = NC * NS

GATHER = 128
CHUNK = 256
SUPER = 1024
LANES = 16


def _rsqrt_scalar(v):
  i = lax.bitcast_convert_type(v, jnp.int32)
  y = lax.bitcast_convert_type(
      jnp.int32(0x5F3759DF) - lax.shift_right_logical(i, 1), jnp.float32)
  for _ in range(3):
    y = y * (1.5 - 0.5 * v * y * y)
  return y


def _tree_sum(vs):
  while len(vs) > 1:
    vs = [a + b for a, b in zip(vs[::2], vs[1::2])]
  return vs[0]


def _make_kernel(total_rows):
  per_w = total_rows // NW
  assert per_w % SUPER == 0
  n_chunks = per_w // CHUNK
  chunks_per_super = SUPER // CHUNK
  g_per_chunk = CHUNK // GATHER

  def body(x_hbm, table_hbm, gamma_hbm, beta_hbm, out_hbm,
           idx_v, rows_a, rows_b, gamma_v, beta_v, gsa, gsb, ssa, ssb):
    wid = lax.axis_index("s") * NC + lax.axis_index("c")
    pltpu.sync_copy(gamma_hbm, gamma_v)
    pltpu.sync_copy(beta_hbm, beta_v)
    base_w = wid * per_w
    lane_iota = lax.iota(jnp.int32, LANES)

    def fetch_idx(si):
      xrow = pl.multiple_of((base_w + si * SUPER) // GATHER, SUPER // GATHER)
      pltpu.sync_copy(x_hbm.at[pl.ds(xrow, SUPER // GATHER)], idx_v)

    def start_gather(c, rows, gsem):
      irow = lax.rem(c, chunks_per_super) * g_per_chunk
      for g in range(g_per_chunk):
        pltpu.async_copy(
            table_hbm.at[idx_v.at[irow + g]],
            rows.at[pl.ds(g * GATHER, GATHER)], gsem)

    def drain_gather(rows, gsem):
      for g in range(g_per_chunk):
        pltpu.make_async_copy(
            table_hbm.at[idx_v.at[g]],
            rows.at[pl.ds(g * GATHER, GATHER)], gsem).wait()

    def start_scatter(c, rows, ssem):
      pltpu.async_copy(rows, out_hbm.at[pl.ds(base_w + c * CHUNK, CHUNK)],
                       ssem)

    def drain_scatter(rows, ssem):
      pltpu.make_async_copy(rows, out_hbm.at[pl.ds(base_w, CHUNK)],
                            ssem).wait()

    g_regs = [gamma_v[pl.ds(LANES * k, LANES)] for k in range(D // LANES)]
    b_regs = [beta_v[pl.ds(LANES * k, LANES)] for k in range(D // LANES)]

    def compute(rows):
      @plsc.parallel_loop(0, CHUNK, unroll=2)
      def row_body(r):
        e = [rows[r, pl.ds(LANES * k, LANES)] for k in range(D // LANES)]
        tot = jnp.sum(_tree_sum(e))
        tot2 = jnp.sum(_tree_sum([ek * ek for ek in e]))
        mean = tot * (1.0 / D)
        var = jnp.maximum(tot2 * (1.0 / D) - mean * mean, 0.0)
        rstd = _rsqrt_scalar(var + EPS_FOLDED)
        shift = -mean * rstd
        for k in range(D // LANES):
          rows[r, pl.ds(LANES * k, LANES)] = (
              (e[k] * rstd + shift) * g_regs[k] + b_regs[k])

    fetch_idx(0)
    start_gather(0, rows_a, gsa)

    def pair_body(p, carry):
      c0 = 2 * p
      c1 = c0 + 1
      c2 = c0 + 2

      @pl.when(p > 0)
      def _():
        drain_scatter(rows_b, ssb)
      start_gather(c1, rows_b, gsb)
      drain_gather(rows_a, gsa)
      compute(rows_a)
      start_scatter(c0, rows_a, ssa)

      drain_gather(rows_b, gsb)

      @pl.when(c2 < n_chunks)
      def _():
        @pl.when(lax.rem(c2, chunks_per_super) == 0)
        def _():
          fetch_idx(c2 // chunks_per_super)
        drain_scatter(rows_a, ssa)
        start_gather(c2, rows_a, gsa)
      compute(rows_b)
      start_scatter(c1, rows_b, ssb)
      return carry

    lax.fori_loop(0, n_chunks // 2, pair_body, 0)
    drain_scatter(rows_a, ssa)
    drain_scatter(rows_b, ssb)

  return body


@jax.jit
def kernel(x, table, gamma, beta):
  bsz, seq = x.shape
  total = bsz * seq
  x2 = x.reshape(total // GATHER, GATHER)
  run = pl.kernel(
      _make_kernel(total),
      out_type=jax.ShapeDtypeStruct((total, D), jnp.float32),
      mesh=plsc.VectorSubcoreMesh(core_axis_name="c", subcore_axis_name="s"),
      compiler_params=pltpu.CompilerParams(needs_layout_passes=False),
      scratch_types=[
          pltpu.VMEM((SUPER // GATHER, GATHER), jnp.int32),
          pltpu.VMEM((CHUNK, D), jnp.float32),
          pltpu.VMEM((CHUNK, D), jnp.float32),
          pltpu.VMEM((D,), jnp.float32),
          pltpu.VMEM((D,), jnp.float32),
          pltpu.SemaphoreType.DMA,
          pltpu.SemaphoreType.DMA,
          pltpu.SemaphoreType.DMA,
          pltpu.SemaphoreType.DMA,
      ],
  )
  out = run(x2, table, gamma, beta)
  return out.reshape(bsz, seq, D)
